# Initial kernel scaffold; baseline (speedup 1.0000x reference)
import jax
import jax.numpy as jnp
from jax.experimental import pallas as pl


def kernel(predicted_locs, predicted_scores, boxes, labels, priors_cxcy):
    raise NotImplementedError("write your pallas kernel here")


# speedup vs baseline: 17.1676x; 17.1676x over previous
import jax
import jax.numpy as jnp
from jax.experimental import pallas as pl

_TH1, _TH2, _TH3 = 0.1, 0.35, 0.5
_NEG_POS_RATIO = 3


def _mbl_kernel(scores_ref, locs_ref, boxes_ref, labels_ref, priors_ref, out_ref):
    C, P = scores_ref.shape[1], scores_ref.shape[2]
    NOBJ = boxes_ref.shape[1]

    pr = priors_ref[...]
    pcx, pcy = pr[0:1, :], pr[1:2, :]
    pw, ph = pr[2:3, :], pr[3:4, :]
    px0, py0 = pcx - pw * 0.5, pcy - ph * 0.5
    px1, py1 = pcx + pw * 0.5, pcy + ph * 0.5

    b = boxes_ref[0]
    bx0, by0 = b[:, 0:1], b[:, 1:2]
    bx1, by1 = b[:, 2:3], b[:, 3:4]

    iw = jnp.clip(jnp.minimum(bx1, px1) - jnp.maximum(bx0, px0), 0.0, None)
    ih = jnp.clip(jnp.minimum(by1, py1) - jnp.maximum(by0, py0), 0.0, None)
    inter = iw * ih
    a1 = (bx1 - bx0) * (by1 - by0)
    a2 = (px1 - px0) * (py1 - py0)
    iou = inter / (a1 + a2 - inter)

    obj_iota = jax.lax.broadcasted_iota(jnp.int32, (NOBJ, P), 0)
    lane_iota = jax.lax.broadcasted_iota(jnp.int32, (NOBJ, P), 1)
    piota = jax.lax.broadcasted_iota(jnp.int32, (1, P), 1)

    ofp = jnp.max(iou, axis=0, keepdims=True)
    obj_fp = jnp.min(jnp.where(iou == ofp, obj_iota, NOBJ),
                     axis=0, keepdims=True)

    ofo = jnp.max(iou, axis=1, keepdims=True)
    pfo = jnp.min(jnp.where(iou == ofo, lane_iota, P),
                  axis=1, keepdims=True)

    pfo_mask = lane_iota == pfo
    mj = jnp.max(jnp.where(pfo_mask, obj_iota, -1), axis=0, keepdims=True)
    obj_fp = jnp.where(mj >= 0, mj, obj_fp)
    ofp = jnp.where(mj >= 0, 2.0, ofp)

    lab = jnp.zeros((1, P), jnp.int32)
    sx0 = jnp.zeros((1, P), jnp.float32)
    sy0 = jnp.zeros((1, P), jnp.float32)
    sx1 = jnp.zeros((1, P), jnp.float32)
    sy1 = jnp.zeros((1, P), jnp.float32)
    for j in range(NOBJ):
        sel = obj_fp == j
        lab = jnp.where(sel, labels_ref[0, 0, j], lab)
        sx0 = jnp.where(sel, b[j, 0], sx0)
        sy0 = jnp.where(sel, b[j, 1], sy0)
        sx1 = jnp.where(sel, b[j, 2], sx1)
        sy1 = jnp.where(sel, b[j, 3], sy1)
    lab = jnp.where(ofp < _TH2, 0, lab)

    n_th2 = jnp.sum((ofo >= _TH2).astype(jnp.int32))
    n_th3 = jnp.sum((ofo >= _TH3).astype(jnp.int32))
    nn = (n_th2 + n_th3) // 2
    clone = jnp.where((ofp > _TH1) & (ofp < _TH2), ofp, 0.0)
    cnt = jnp.sum((clone > _TH1).astype(jnp.int32))
    n_add = jnp.minimum(cnt, nn)
    for t in range(NOBJ):
        m = jnp.max(clone)
        idx = jnp.min(jnp.where(clone == m, piota, P))
        onehot = piota == idx
        lab = jnp.where(onehot & (t < n_add), lab + 1, lab)
        clone = jnp.where(onehot, -1.0, clone)

    bcx, bcy = (sx0 + sx1) * 0.5, (sy0 + sy1) * 0.5
    bw, bh = sx1 - sx0, sy1 - sy0
    tgx = (bcx - pcx) * 10.0 / pw
    tgy = (bcy - pcy) * 10.0 / ph
    tgw = jnp.log(bw / pw) * 5.0
    tgh = jnp.log(bh / ph) * 5.0

    pos = lab != 0
    n_pos = jnp.sum(pos.astype(jnp.int32))

    l = locs_ref[0]
    posf = pos.astype(jnp.float32)
    loc_abs = (jnp.sum(jnp.abs(l[0:1] - tgx) * posf)
               + jnp.sum(jnp.abs(l[1:2] - tgy) * posf)
               + jnp.sum(jnp.abs(l[2:3] - tgw) * posf)
               + jnp.sum(jnp.abs(l[3:4] - tgh) * posf))

    s = scores_ref[0]
    smax = jnp.max(s, axis=0, keepdims=True)
    lse = smax + jnp.log(jnp.sum(jnp.exp(s - smax), axis=0, keepdims=True))
    ciota = jax.lax.broadcasted_iota(jnp.int32, (C, P), 0)
    s_true = jnp.sum(jnp.where(ciota == lab, s, 0.0), axis=0, keepdims=True)
    ce = lse - s_true
    sum_pos_ce = jnp.sum(jnp.where(pos, ce, 0.0))
    neg = jnp.where(pos, 0.0, ce)

    kk = _NEG_POS_RATIO * n_pos
    bits = jax.lax.bitcast_convert_type(neg, jnp.uint32)

    def _bit_step(i, pivot):
        t = pivot | (jnp.uint32(1) << (jnp.uint32(31) - i.astype(jnp.uint32)))
        c = jnp.sum((bits >= t).astype(jnp.int32))
        return jnp.where(c >= kk, t, pivot)

    pivot = jax.lax.fori_loop(0, 32, _bit_step, jnp.uint32(0))
    gt = bits > pivot
    cnt_gt = jnp.sum(gt.astype(jnp.int32))
    sum_gt = jnp.sum(jnp.where(gt, neg, 0.0))
    pivot_f = jax.lax.bitcast_convert_type(pivot, jnp.float32)
    topk = jnp.where(kk > 0,
                     sum_gt + (kk - cnt_gt).astype(jnp.float32) * pivot_f,
                     0.0)

    o_iota = jax.lax.broadcasted_iota(jnp.int32, (1, 1, 8), 2)
    row = (jnp.where(o_iota == 0, n_pos.astype(jnp.float32), 0.0)
           + jnp.where(o_iota == 1, sum_pos_ce, 0.0)
           + jnp.where(o_iota == 2, topk, 0.0)
           + jnp.where(o_iota == 3, loc_abs, 0.0))
    out_ref[...] = row


def kernel(predicted_locs, predicted_scores, boxes, labels, priors_cxcy):
    B, P, C = predicted_scores.shape
    NOBJ = boxes.shape[1]
    scores_t = jnp.transpose(predicted_scores, (0, 2, 1))
    locs_t = jnp.transpose(predicted_locs, (0, 2, 1))
    priors_t = jnp.transpose(priors_cxcy, (1, 0))
    labels_r = labels.astype(jnp.int32).reshape(B, 1, NOBJ)

    parts = pl.pallas_call(
        _mbl_kernel,
        grid=(B,),
        in_specs=[
            pl.BlockSpec((1, C, P), lambda b: (b, 0, 0)),
            pl.BlockSpec((1, 4, P), lambda b: (b, 0, 0)),
            pl.BlockSpec((1, NOBJ, 4), lambda b: (b, 0, 0)),
            pl.BlockSpec((1, 1, NOBJ), lambda b: (b, 0, 0)),
            pl.BlockSpec((4, P), lambda b: (0, 0)),
        ],
        out_specs=pl.BlockSpec((1, 1, 8), lambda b: (b, 0, 0)),
        out_shape=jax.ShapeDtypeStruct((B, 1, 8), jnp.float32),
    )(scores_t, locs_t, boxes, labels_r, priors_t)
    parts = parts[:, 0, :]

    n_pos_tot = jnp.sum(parts[:, 0])
    conf_loss = (jnp.sum(parts[:, 1]) + jnp.sum(parts[:, 2])) / n_pos_tot
    loc_loss = jnp.sum(parts[:, 3]) / (n_pos_tot * 4.0)
    return conf_loss, loc_loss


# speedup vs baseline: 32.3301x; 1.8832x over previous
import jax
import jax.numpy as jnp
from jax.experimental import pallas as pl
from jax.experimental.pallas import tpu as pltpu

_TH1, _TH2, _TH3 = 0.1, 0.35, 0.5
_NEG_POS_RATIO = 3


def _mbl_kernel(scores_ref, locs_ref, boxes_ref, labels_ref, priors_ref, out_ref, *, p_true):
    C, R, PW = scores_ref.shape[1], scores_ref.shape[2], scores_ref.shape[3]
    NOBJ = boxes_ref.shape[1]
    P8 = R * PW

    pr = priors_ref[...]
    pcx, pcy, pw, ph = pr[0], pr[1], pr[2], pr[3]
    px0, py0 = pcx - pw * 0.5, pcy - ph * 0.5
    px1, py1 = pcx + pw * 0.5, pcy + ph * 0.5

    b = boxes_ref[0]
    bx0 = b[:, 0].reshape(NOBJ, 1, 1)
    by0 = b[:, 1].reshape(NOBJ, 1, 1)
    bx1 = b[:, 2].reshape(NOBJ, 1, 1)
    by1 = b[:, 3].reshape(NOBJ, 1, 1)

    iw = jnp.clip(jnp.minimum(bx1, px1[None]) - jnp.maximum(bx0, px0[None]), 0.0, None)
    ih = jnp.clip(jnp.minimum(by1, py1[None]) - jnp.maximum(by0, py0[None]), 0.0, None)
    inter = iw * ih
    a1 = (bx1 - bx0) * (by1 - by0)
    a2 = ((px1 - px0) * (py1 - py0))[None]
    iou = inter / (a1 + a2 - inter)

    obj_iota = jax.lax.broadcasted_iota(jnp.int32, (NOBJ, R, PW), 0)
    pidx3 = (jax.lax.broadcasted_iota(jnp.int32, (NOBJ, R, PW), 1) * PW
             + jax.lax.broadcasted_iota(jnp.int32, (NOBJ, R, PW), 2))
    pidx = (jax.lax.broadcasted_iota(jnp.int32, (R, PW), 0) * PW
            + jax.lax.broadcasted_iota(jnp.int32, (R, PW), 1))
    valid = pidx < p_true

    ofp = jnp.max(iou, axis=0)
    obj_fp = jnp.min(jnp.where(iou == ofp[None], obj_iota, NOBJ), axis=0)

    ofo = jnp.max(iou, axis=(1, 2), keepdims=True)
    pfo = jnp.min(jnp.where(iou == ofo, pidx3, P8),
                  axis=(1, 2), keepdims=True)

    pfo_mask = pidx3 == pfo
    mj = jnp.max(jnp.where(pfo_mask, obj_iota, -1), axis=0)
    obj_fp = jnp.where(mj >= 0, mj, obj_fp)
    ofp = jnp.where(mj >= 0, 2.0, ofp)

    lab = jnp.zeros((R, PW), jnp.int32)
    sx0 = jnp.zeros((R, PW), jnp.float32)
    sy0 = jnp.zeros((R, PW), jnp.float32)
    sx1 = jnp.zeros((R, PW), jnp.float32)
    sy1 = jnp.zeros((R, PW), jnp.float32)
    for j in range(NOBJ):
        sel = obj_fp == j
        lab = jnp.where(sel, labels_ref[0, 0, j], lab)
        sx0 = jnp.where(sel, b[j, 0], sx0)
        sy0 = jnp.where(sel, b[j, 1], sy0)
        sx1 = jnp.where(sel, b[j, 2], sx1)
        sy1 = jnp.where(sel, b[j, 3], sy1)
    lab = jnp.where(ofp < _TH2, 0, lab)

    n_th2 = jnp.sum((ofo >= _TH2).astype(jnp.int32))
    n_th3 = jnp.sum((ofo >= _TH3).astype(jnp.int32))
    nn = (n_th2 + n_th3) // 2
    clone = jnp.where((ofp > _TH1) & (ofp < _TH2), ofp, 0.0)
    cnt = jnp.sum((clone > _TH1).astype(jnp.int32))
    n_add = jnp.minimum(cnt, nn)
    for t in range(NOBJ):
        m = jnp.max(clone)
        idx = jnp.min(jnp.where(clone == m, pidx, P8))
        onehot = pidx == idx
        lab = jnp.where(onehot & (t < n_add), lab + 1, lab)
        clone = jnp.where(onehot, -1.0, clone)

    bcx, bcy = (sx0 + sx1) * 0.5, (sy0 + sy1) * 0.5
    bw, bh = sx1 - sx0, sy1 - sy0
    tgx = (bcx - pcx) * 10.0 / pw
    tgy = (bcy - pcy) * 10.0 / ph
    tgw = jnp.log(bw / pw) * 5.0
    tgh = jnp.log(bh / ph) * 5.0

    pos = lab != 0
    n_pos = jnp.sum(pos.astype(jnp.int32))

    l = locs_ref[0]
    posf = pos.astype(jnp.float32)
    loc_abs = (jnp.sum(jnp.abs(l[0] - tgx) * posf)
               + jnp.sum(jnp.abs(l[1] - tgy) * posf)
               + jnp.sum(jnp.abs(l[2] - tgw) * posf)
               + jnp.sum(jnp.abs(l[3] - tgh) * posf))

    s = scores_ref[0]
    smax = jnp.max(s, axis=0)
    lse = smax + jnp.log(jnp.sum(jnp.exp(s - smax[None]), axis=0))
    ciota = jax.lax.broadcasted_iota(jnp.int32, (C, R, PW), 0)
    s_true = jnp.sum(jnp.where(ciota == lab[None], s, 0.0), axis=0)
    ce = lse - s_true
    sum_pos_ce = jnp.sum(jnp.where(pos, ce, 0.0))
    neg = jnp.where(pos | ~valid, 0.0, ce)

    kk = _NEG_POS_RATIO * n_pos
    bits = jax.lax.bitcast_convert_type(neg, jnp.uint32)

    def _bit_step(i, pivot):
        t = pivot | (jnp.uint32(1) << (jnp.uint32(31) - i.astype(jnp.uint32)))
        c = jnp.sum((bits >= t).astype(jnp.int32))
        return jnp.where(c >= kk, t, pivot)

    pivot = jax.lax.fori_loop(1, 32, _bit_step, jnp.uint32(0))
    gt = bits > pivot
    cnt_gt = jnp.sum(gt.astype(jnp.int32))
    sum_gt = jnp.sum(jnp.where(gt, neg, 0.0))
    pivot_f = jax.lax.bitcast_convert_type(pivot, jnp.float32)
    topk = jnp.where(kk > 0,
                     sum_gt + (kk - cnt_gt).astype(jnp.float32) * pivot_f,
                     0.0)

    o_iota = jax.lax.broadcasted_iota(jnp.int32, (1, 1, 8), 2)
    row = (jnp.where(o_iota == 0, n_pos.astype(jnp.float32), 0.0)
           + jnp.where(o_iota == 1, sum_pos_ce, 0.0)
           + jnp.where(o_iota == 2, topk, 0.0)
           + jnp.where(o_iota == 3, loc_abs, 0.0))
    out_ref[...] = row


def kernel(predicted_locs, predicted_scores, boxes, labels, priors_cxcy):
    B, P, C = predicted_scores.shape
    NOBJ = boxes.shape[1]
    R = 8
    P8 = ((P + R - 1) // R) * R
    PW = P8 // R
    pad = P8 - P

    scores_t = jnp.transpose(predicted_scores, (0, 2, 1))
    scores_t = jnp.pad(scores_t, ((0, 0), (0, 0), (0, pad))).reshape(B, C, R, PW)
    locs_t = jnp.transpose(predicted_locs, (0, 2, 1))
    locs_t = jnp.pad(locs_t, ((0, 0), (0, 0), (0, pad))).reshape(B, 4, R, PW)
    priors_t = jnp.transpose(priors_cxcy, (1, 0))
    priors_t = jnp.pad(priors_t, ((0, 0), (0, pad))).reshape(4, R, PW)
    labels_r = labels.astype(jnp.int32).reshape(B, 1, NOBJ)

    import functools
    parts = pl.pallas_call(
        functools.partial(_mbl_kernel, p_true=P),
        grid=(B,),
        in_specs=[
            pl.BlockSpec((1, C, R, PW), lambda b: (b, 0, 0, 0)),
            pl.BlockSpec((1, 4, R, PW), lambda b: (b, 0, 0, 0)),
            pl.BlockSpec((1, NOBJ, 4), lambda b: (b, 0, 0)),
            pl.BlockSpec((1, 1, NOBJ), lambda b: (b, 0, 0)),
            pl.BlockSpec((4, R, PW), lambda b: (0, 0, 0)),
        ],
        out_specs=pl.BlockSpec((1, 1, 8), lambda b: (b, 0, 0)),
        out_shape=jax.ShapeDtypeStruct((B, 1, 8), jnp.float32),
        compiler_params=pltpu.CompilerParams(
            dimension_semantics=("parallel",)),
    )(scores_t, locs_t, boxes, labels_r, priors_t)
    parts = parts[:, 0, :]

    n_pos_tot = jnp.sum(parts[:, 0])
    conf_loss = (jnp.sum(parts[:, 1]) + jnp.sum(parts[:, 2])) / n_pos_tot
    loc_loss = jnp.sum(parts[:, 3]) / (n_pos_tot * 4.0)
    return conf_loss, loc_loss
